# Initial kernel scaffold; baseline (speedup 1.0000x reference)
import jax
import jax.numpy as jnp
from jax.experimental import pallas as pl


def kernel(x, edge_index, W_src, W_dst, W_src2, W_dst2, attn_l, attn_r):
    raise NotImplementedError("write your pallas kernel here")


# speedup vs baseline: 9.2445x; 9.2445x over previous
import functools

import jax
import jax.numpy as jnp
import numpy as np
from jax import lax
from jax.experimental import pallas as pl
from jax.experimental.pallas import tpu as pltpu
from jax.experimental.pallas import tpu_sc as plsc

L = 16
NSUB = 16
NCORE = 2
EB = 2000
NEG_BIG = -3.0e38


def _mm_body(x_ref, ws_ref, ws2_ref, wd_ref, wd2_ref, al_ref, ar_ref,
             f1_ref, f2_ref, fd1_ref, el_ref, er_ref):
    x = x_ref[...]
    dn = (((1,), (1,)), ((), ()))
    f1 = lax.dot_general(x, ws_ref[...], dn, preferred_element_type=jnp.float32)
    f2 = lax.dot_general(x, ws2_ref[...], dn, preferred_element_type=jnp.float32)
    fd1 = lax.dot_general(x, wd_ref[...], dn, preferred_element_type=jnp.float32)
    fd2 = lax.dot_general(x, wd2_ref[...], dn, preferred_element_type=jnp.float32)
    f1_ref[...] = f1
    f2_ref[...] = f2
    fd1_ref[...] = fd1
    el_ref[...] = lax.dot_general(f2, al_ref[...], dn,
                                  preferred_element_type=jnp.float32)
    er_ref[...] = lax.dot_general(fd2, ar_ref[...], dn,
                                  preferred_element_type=jnp.float32)


def _dense_stage(x, W_src, W_src2, W_dst, W_dst2, attn_l, attn_r):
    n, d = x.shape
    m = 1000
    bs_x = pl.BlockSpec((m, d), lambda i: (i, 0))
    bs_w = pl.BlockSpec((d, d), lambda i: (0, 0))
    bs_a = pl.BlockSpec((1, d), lambda i: (0, 0))
    bs_o = pl.BlockSpec((m, d), lambda i: (i, 0))
    bs_s = pl.BlockSpec((m, 1), lambda i: (i, 0))
    f32 = jnp.float32
    return pl.pallas_call(
        _mm_body,
        grid=(n // m,),
        in_specs=[bs_x, bs_w, bs_w, bs_w, bs_w, bs_a, bs_a],
        out_specs=[bs_o, bs_o, bs_o, bs_s, bs_s],
        out_shape=[jax.ShapeDtypeStruct((n, d), f32),
                   jax.ShapeDtypeStruct((n, d), f32),
                   jax.ShapeDtypeStruct((n, d), f32),
                   jax.ShapeDtypeStruct((n, 1), f32),
                   jax.ShapeDtypeStruct((n, 1), f32)],
    )(x, W_src, W_src2, W_dst, W_dst2, attn_l, attn_r)


def _add3_body(a_ref, b_ref, c_ref, o_ref):
    o_ref[...] = a_ref[...] + b_ref[...] + c_ref[...]


def _final_stage(fd1, p0, p1):
    n, d = fd1.shape
    m = 1000
    bs = pl.BlockSpec((m, d), lambda i: (i, 0))
    return pl.pallas_call(
        _add3_body,
        grid=(n // m,),
        in_specs=[bs, bs, bs],
        out_specs=bs,
        out_shape=jax.ShapeDtypeStruct((n, d), jnp.float32),
    )(fd1, p0, p1)


def _leaky(v):
    return jnp.where(v >= 0.0, v, 0.2 * v)


_GDN = lax.GatherDimensionNumbers(offset_dims=(), collapsed_slice_dims=(0,),
                                  start_index_map=(0,))


def _lane_gather(v, idx):
    return lax.gather(v, idx[:, None], _GDN, (1,),
                      mode=lax.GatherScatterMode.PROMISE_IN_BOUNDS)


def _lane_iota():
    return lax.iota(jnp.int32, L)


def _group_max(d16, ev):
    lane = _lane_iota()
    gmax = ev
    for k in range(1, L):
        rot = lax.rem(lane + k, L)
        eq = _lane_gather(d16, rot) == d16
        gmax = jnp.where(eq, jnp.maximum(gmax, _lane_gather(ev, rot)), gmax)
    return gmax


def _group_sum_first(d16, ev):
    lane = _lane_iota()
    gsum = ev
    first = lane >= 0
    for k in range(1, L):
        rot = lax.rem(lane + k, L)
        eq = _lane_gather(d16, rot) == d16
        gsum = gsum + jnp.where(eq, _lane_gather(ev, rot), 0.0)
        sh = jnp.maximum(lane - k, 0)
        dup = (_lane_gather(d16, sh) == d16) & (lane >= k)
        first = first & jnp.logical_not(dup)
    return gsum, first


def _make_e_stage(n, e_cnt, np_):
    f32 = jnp.float32
    i32 = jnp.int32
    ek = e_cnt // (NSUB * NCORE)
    mesh = plsc.VectorSubcoreMesh(core_axis_name="c", subcore_axis_name="s")

    @functools.partial(
        pl.kernel, mesh=mesh,
        compiler_params=pltpu.CompilerParams(needs_layout_passes=False),
        out_type=jax.ShapeDtypeStruct((e_cnt,), f32),
        scratch_types=[
            pltpu.VMEM((np_,), f32),
            pltpu.VMEM((np_,), f32),
            pltpu.VMEM((EB,), i32),
            pltpu.VMEM((EB,), i32),
            pltpu.VMEM((EB,), f32),
        ])
    def e_stage(src_hbm, dst_hbm, el_hbm, er_hbm, e_out_hbm,
                el_v, er_v, sblk_v, dblk_v, eblk_v):
        c = lax.axis_index("c")
        s = lax.axis_index("s")
        wid = s * NCORE + c
        pltpu.sync_copy(el_hbm, el_v)
        pltpu.sync_copy(er_hbm, er_v)

        def blk(b, _):
            base = wid * ek + b * EB
            pltpu.sync_copy(src_hbm.at[pl.ds(base, EB)], sblk_v)
            pltpu.sync_copy(dst_hbm.at[pl.ds(base, EB)], dblk_v)

            def inner(i, _):
                s16 = sblk_v[pl.ds(i * L, L)]
                d16 = dblk_v[pl.ds(i * L, L)]
                eblk_v[pl.ds(i * L, L)] = _leaky(
                    plsc.load_gather(el_v, [s16])
                    + plsc.load_gather(er_v, [d16]))
                return 0
            lax.fori_loop(0, EB // L, inner, 0)
            pltpu.sync_copy(eblk_v, e_out_hbm.at[pl.ds(base, EB)])
            return 0
        lax.fori_loop(0, ek // EB, blk, 0)

    return e_stage


def _make_sc_stage(n, e_cnt, d, np_, sw, ea, ec):
    f32 = jnp.float32
    i32 = jnp.int32
    mesh = plsc.VectorSubcoreMesh(core_axis_name="c", subcore_axis_name="s")

    @functools.partial(
        pl.kernel, mesh=mesh,
        compiler_params=pltpu.CompilerParams(needs_layout_passes=False),
        out_type=[jax.ShapeDtypeStruct((np_, d), f32),
                  jax.ShapeDtypeStruct((np_, d), f32),
                  jax.ShapeDtypeStruct((e_cnt,), f32)],
        scratch_types=[
            pltpu.VMEM((np_,), f32),
            pltpu.VMEM((np_,), f32),
            pltpu.VMEM((EB,), i32),
            pltpu.VMEM((EB,), i32),
            pltpu.VMEM((EB,), f32),
            pltpu.VMEM((sw,), f32),
            pltpu.VMEM((sw,), f32),
            pltpu.VMEM((L,), i32),
            pltpu.VMEM((L,), i32),
            pltpu.VMEM((L, d), f32),
            pltpu.VMEM((L, d), f32),
            pltpu.VMEM((L, d), f32),
            pltpu.VMEM_SHARED((NSUB, np_), f32),
            pltpu.VMEM_SHARED((np_,), f32),
            pltpu.VMEM_SHARED((np_, d), f32),
            pltpu.SemaphoreType.DMA,
            pltpu.SemaphoreType.DMA,
        ])
    def sc_stage(src_hbm, dst_hbm, e_hbm, f1_hbm, f2_hbm,
                 p0_hbm, p1_hbm, ex_hbm,
                 emax_v, denom_v, sblk_v, dblk_v, eblk_v,
                 strip_v, racc_v, sidx16_v, didx16_v,
                 rows1_v, rows2_v, zrow_v,
                 shared_red, shared_vec, acc_sh, sem1, sem2):
        c = lax.axis_index("c")
        s = lax.axis_index("s")

        neg = jnp.full((L,), NEG_BIG, f32)
        zero = jnp.zeros((L,), f32)

        def init_body(i, _):
            emax_v[pl.ds(i * L, L)] = neg
            denom_v[pl.ds(i * L, L)] = zero
            return 0
        lax.fori_loop(0, np_ // L, init_body, 0)
        sidx16_v[...] = jnp.zeros((L,), i32)
        didx16_v[...] = jnp.zeros((L,), i32)
        for r in range(L):
            for f0 in range(d // L):
                zrow_v[r, pl.ds(f0 * L, L)] = zero

        def phase_a_blk(b, _):
            base = s * ea + b * EB
            pltpu.sync_copy(dst_hbm.at[pl.ds(base, EB)], dblk_v)
            pltpu.sync_copy(e_hbm.at[pl.ds(base, EB)], eblk_v)

            def inner(i, _):
                d16 = dblk_v[pl.ds(i * L, L)]
                ev = eblk_v[pl.ds(i * L, L)]
                gmax = _group_max(d16, ev)
                cur = plsc.load_gather(emax_v, [d16])
                plsc.store_scatter(emax_v, [d16], jnp.maximum(cur, gmax))
                return 0
            lax.fori_loop(0, EB // L, inner, 0)
            return 0
        lax.fori_loop(0, ea // EB, phase_a_blk, 0)

        def stripe_reduce(node_v, op):
            pltpu.sync_copy(node_v, shared_red.at[s])
            plsc.subcore_barrier()
            pltpu.sync_copy(shared_red.at[0, pl.ds(s * sw, sw)], racc_v)
            for k in range(1, NSUB):
                pltpu.sync_copy(shared_red.at[k, pl.ds(s * sw, sw)], strip_v)

                def red_body(i, _):
                    sl = pl.ds(i * L, L)
                    racc_v[sl] = op(racc_v[sl], strip_v[sl])
                    return 0
                lax.fori_loop(0, sw // L, red_body, 0)
            pltpu.sync_copy(racc_v, shared_vec.at[pl.ds(s * sw, sw)])
            plsc.subcore_barrier()
            pltpu.sync_copy(shared_vec, node_v)

        stripe_reduce(emax_v, jnp.maximum)

        for k in range(sw // L):
            pltpu.sync_copy(zrow_v, acc_sh.at[pl.ds(s * sw + k * L, L), :])

        def phase_b_blk(b, _):
            base = s * ea + b * EB
            pltpu.sync_copy(dst_hbm.at[pl.ds(base, EB)], dblk_v)
            pltpu.sync_copy(e_hbm.at[pl.ds(base, EB)], eblk_v)

            def inner(i, _):
                d16 = dblk_v[pl.ds(i * L, L)]
                em = plsc.load_gather(emax_v, [d16])
                ex = jnp.exp(eblk_v[pl.ds(i * L, L)] - em)
                eblk_v[pl.ds(i * L, L)] = ex
                gsum, first = _group_sum_first(d16, ex)
                cur = plsc.load_gather(denom_v, [d16])
                plsc.store_scatter(denom_v, [d16], cur + gsum, mask=first)
                return 0
            lax.fori_loop(0, EB // L, inner, 0)
            pltpu.sync_copy(eblk_v, ex_hbm.at[pl.ds(base, EB)])
            return 0
        lax.fori_loop(0, ea // EB, phase_b_blk, 0)

        stripe_reduce(denom_v, jnp.add)

        def phase_c_blk(b, _):
            base = s * ea + c * ec + b * EB
            pltpu.sync_copy(src_hbm.at[pl.ds(base, EB)], sblk_v)
            pltpu.sync_copy(dst_hbm.at[pl.ds(base, EB)], dblk_v)
            pltpu.sync_copy(ex_hbm.at[pl.ds(base, EB)], eblk_v)

            def inner(i, _):
                s16 = sblk_v[pl.ds(i * L, L)]
                d16 = dblk_v[pl.ds(i * L, L)]
                sidx16_v[...] = s16
                didx16_v[...] = d16
                dn = plsc.load_gather(denom_v, [d16])
                av = eblk_v[pl.ds(i * L, L)] / dn
                pltpu.async_copy(f1_hbm.at[sidx16_v], rows1_v, sem1).wait()
                pltpu.async_copy(f2_hbm.at[sidx16_v], rows2_v, sem2).wait()
                for r in range(L):
                    asc = av[r]
                    for f0 in range(d // L):
                        sl = pl.ds(f0 * L, L)
                        rows2_v[r, sl] = rows2_v[r, sl] * asc
                pltpu.sync_copy(rows1_v, acc_sh.at[didx16_v], add=True)
                pltpu.sync_copy(rows2_v, acc_sh.at[didx16_v], add=True)
                return 0
            lax.fori_loop(0, EB // L, inner, 0)
            return 0
        lax.fori_loop(0, ec // EB, phase_c_blk, 0)

        plsc.subcore_barrier()

        @pl.when(c == 0)
        def _():
            pltpu.sync_copy(acc_sh.at[pl.ds(s * sw, sw), :],
                            p0_hbm.at[pl.ds(s * sw, sw), :])

        @pl.when(c == 1)
        def _():
            pltpu.sync_copy(acc_sh.at[pl.ds(s * sw, sw), :],
                            p1_hbm.at[pl.ds(s * sw, sw), :])

    return sc_stage


def kernel(x, edge_index, W_src, W_dst, W_src2, W_dst2, attn_l, attn_r):
    n, d = x.shape
    e_cnt = edge_index.shape[1]
    blk = NSUB * L
    np_ = ((n + blk - 1) // blk) * blk
    sw = np_ // NSUB
    ea = e_cnt // NSUB
    ec = e_cnt // (NSUB * NCORE)

    f1, f2, fd1, el, er = _dense_stage(x, W_src, W_src2, W_dst, W_dst2,
                                       attn_l, attn_r)
    src = edge_index[0]
    dst = edge_index[1]
    e_stage = _make_e_stage(n, e_cnt, np_)
    el_p = jnp.pad(el.reshape(-1), (0, np_ - n))
    er_p = jnp.pad(er.reshape(-1), (0, np_ - n))
    e_vals = e_stage(src, dst, el_p, er_p)
    sc_stage = _make_sc_stage(n, e_cnt, d, np_, sw, ea, ec)
    p0, p1, _ex = sc_stage(src, dst, e_vals, f1, f2)
    return _final_stage(fd1, p0[:n], p1[:n])


# speedup vs baseline: 14.8090x; 1.6019x over previous
import functools

import jax
import jax.numpy as jnp
import numpy as np
from jax import lax
from jax.experimental import pallas as pl
from jax.experimental.pallas import tpu as pltpu
from jax.experimental.pallas import tpu_sc as plsc

L = 16
NSUB = 16
NCORE = 2
EB = 2000
NEG_BIG = -3.0e38


def _mm_body(x_ref, ws_ref, ws2_ref, wd_ref, wd2_ref, al_ref, ar_ref,
             f1_ref, f2_ref, fd1_ref, el_ref, er_ref):
    x = x_ref[...]
    dn = (((1,), (1,)), ((), ()))
    f1 = lax.dot_general(x, ws_ref[...], dn, preferred_element_type=jnp.float32)
    f2 = lax.dot_general(x, ws2_ref[...], dn, preferred_element_type=jnp.float32)
    fd1 = lax.dot_general(x, wd_ref[...], dn, preferred_element_type=jnp.float32)
    fd2 = lax.dot_general(x, wd2_ref[...], dn, preferred_element_type=jnp.float32)
    f1_ref[...] = f1
    f2_ref[...] = f2
    fd1_ref[...] = fd1
    el_ref[...] = lax.dot_general(f2, al_ref[...], dn,
                                  preferred_element_type=jnp.float32)
    er_ref[...] = lax.dot_general(fd2, ar_ref[...], dn,
                                  preferred_element_type=jnp.float32)


def _dense_stage(x, W_src, W_src2, W_dst, W_dst2, attn_l, attn_r):
    n, d = x.shape
    m = 1000
    bs_x = pl.BlockSpec((m, d), lambda i: (i, 0))
    bs_w = pl.BlockSpec((d, d), lambda i: (0, 0))
    bs_a = pl.BlockSpec((1, d), lambda i: (0, 0))
    bs_o = pl.BlockSpec((m, d), lambda i: (i, 0))
    bs_s = pl.BlockSpec((m, 1), lambda i: (i, 0))
    f32 = jnp.float32
    return pl.pallas_call(
        _mm_body,
        grid=(n // m,),
        in_specs=[bs_x, bs_w, bs_w, bs_w, bs_w, bs_a, bs_a],
        out_specs=[bs_o, bs_o, bs_o, bs_s, bs_s],
        out_shape=[jax.ShapeDtypeStruct((n, d), f32),
                   jax.ShapeDtypeStruct((n, d), f32),
                   jax.ShapeDtypeStruct((n, d), f32),
                   jax.ShapeDtypeStruct((n, 1), f32),
                   jax.ShapeDtypeStruct((n, 1), f32)],
    )(x, W_src, W_src2, W_dst, W_dst2, attn_l, attn_r)


def _add3_body(a_ref, b_ref, c_ref, o_ref):
    o_ref[...] = a_ref[...] + b_ref[...] + c_ref[...]


def _final_stage(fd1, p0, p1):
    n, d = fd1.shape
    m = 1000
    bs = pl.BlockSpec((m, d), lambda i: (i, 0))
    return pl.pallas_call(
        _add3_body,
        grid=(n // m,),
        in_specs=[bs, bs, bs],
        out_specs=bs,
        out_shape=jax.ShapeDtypeStruct((n, d), jnp.float32),
    )(fd1, p0, p1)


def _leaky(v):
    return jnp.where(v >= 0.0, v, 0.2 * v)


_GDN = lax.GatherDimensionNumbers(offset_dims=(), collapsed_slice_dims=(0,),
                                  start_index_map=(0,))


def _lane_gather(v, idx):
    return lax.gather(v, idx[:, None], _GDN, (1,),
                      mode=lax.GatherScatterMode.PROMISE_IN_BOUNDS)


def _lane_iota():
    return lax.iota(jnp.int32, L)


def _group_max(d16, ev):
    lane = _lane_iota()
    gmax = ev
    for k in range(1, L):
        rot = lax.rem(lane + k, L)
        eq = _lane_gather(d16, rot) == d16
        gmax = jnp.where(eq, jnp.maximum(gmax, _lane_gather(ev, rot)), gmax)
    return gmax


def _group_sum_first(d16, ev):
    lane = _lane_iota()
    gsum = ev
    first = lane >= 0
    for k in range(1, L):
        rot = lax.rem(lane + k, L)
        eq = _lane_gather(d16, rot) == d16
        gsum = gsum + jnp.where(eq, _lane_gather(ev, rot), 0.0)
        sh = jnp.maximum(lane - k, 0)
        dup = (_lane_gather(d16, sh) == d16) & (lane >= k)
        first = first & jnp.logical_not(dup)
    return gsum, first


def _make_e_stage(n, e_cnt, np_):
    f32 = jnp.float32
    i32 = jnp.int32
    ek = e_cnt // (NSUB * NCORE)
    mesh = plsc.VectorSubcoreMesh(core_axis_name="c", subcore_axis_name="s")

    @functools.partial(
        pl.kernel, mesh=mesh,
        compiler_params=pltpu.CompilerParams(needs_layout_passes=False),
        out_type=jax.ShapeDtypeStruct((e_cnt,), f32),
        scratch_types=[
            pltpu.VMEM((np_,), f32),
            pltpu.VMEM((np_,), f32),
            pltpu.VMEM((EB,), i32),
            pltpu.VMEM((EB,), i32),
            pltpu.VMEM((EB,), f32),
        ])
    def e_stage(src_hbm, dst_hbm, el_hbm, er_hbm, e_out_hbm,
                el_v, er_v, sblk_v, dblk_v, eblk_v):
        c = lax.axis_index("c")
        s = lax.axis_index("s")
        wid = s * NCORE + c
        pltpu.sync_copy(el_hbm, el_v)
        pltpu.sync_copy(er_hbm, er_v)

        def blk(b, _):
            base = wid * ek + b * EB
            pltpu.sync_copy(src_hbm.at[pl.ds(base, EB)], sblk_v)
            pltpu.sync_copy(dst_hbm.at[pl.ds(base, EB)], dblk_v)

            def inner(i, _):
                s16 = sblk_v[pl.ds(i * L, L)]
                d16 = dblk_v[pl.ds(i * L, L)]
                eblk_v[pl.ds(i * L, L)] = _leaky(
                    plsc.load_gather(el_v, [s16])
                    + plsc.load_gather(er_v, [d16]))
                return 0
            lax.fori_loop(0, EB // L, inner, 0)
            pltpu.sync_copy(eblk_v, e_out_hbm.at[pl.ds(base, EB)])
            return 0
        lax.fori_loop(0, ek // EB, blk, 0)

    return e_stage


def _make_sc_stage(n, e_cnt, d, np_, sw, ea, ec):
    f32 = jnp.float32
    i32 = jnp.int32
    mesh = plsc.VectorSubcoreMesh(core_axis_name="c", subcore_axis_name="s")

    B2 = 32
    NB = EB // B2
    TL = EB - NB * B2

    @functools.partial(
        pl.kernel, mesh=mesh,
        compiler_params=pltpu.CompilerParams(needs_layout_passes=False),
        out_type=[jax.ShapeDtypeStruct((np_, d), f32),
                  jax.ShapeDtypeStruct((np_, d), f32),
                  jax.ShapeDtypeStruct((e_cnt,), f32),
                  jax.ShapeDtypeStruct((NCORE * NSUB * np_,), f32),
                  jax.ShapeDtypeStruct((NCORE * np_,), f32)],
        scratch_types=[
            pltpu.VMEM((np_,), f32),
            pltpu.VMEM((np_,), f32),
            pltpu.VMEM((EB,), i32),
            pltpu.VMEM((EB,), i32),
            pltpu.VMEM((EB,), f32),
            pltpu.VMEM((sw,), f32),
            pltpu.VMEM((sw,), f32),
            pltpu.VMEM((B2,), i32),
            pltpu.VMEM((B2,), i32),
            pltpu.VMEM((B2,), i32),
            pltpu.VMEM((B2,), i32),
            pltpu.VMEM((L,), i32),
            pltpu.VMEM((L,), i32),
            pltpu.VMEM((B2, d), f32),
            pltpu.VMEM((B2, d), f32),
            pltpu.VMEM((B2, d), f32),
            pltpu.VMEM((B2, d), f32),
            pltpu.VMEM_SHARED((np_, d), f32),
            pltpu.SemaphoreType.DMA,
            pltpu.SemaphoreType.DMA,
            pltpu.SemaphoreType.DMA,
            pltpu.SemaphoreType.DMA,
        ])
    def sc_stage(src_hbm, dst_hbm, e_hbm, f1_hbm, f2_hbm,
                 p0_hbm, p1_hbm, ex_hbm, red_hbm, res_hbm,
                 emax_v, denom_v, sblk_v, dblk_v, eblk_v,
                 strip_v, racc_v, sidxb0, sidxb1, didxb0, didxb1,
                 sidxt, didxt, rows1a, rows1b, rows2a, rows2b,
                 acc_sh, sem1a, sem2a, sem1b, sem2b):
        c = lax.axis_index("c")
        s = lax.axis_index("s")

        neg = jnp.full((L,), NEG_BIG, f32)
        zero = jnp.zeros((L,), f32)
        zeroi = jnp.zeros((L,), i32)

        def init_body(i, _):
            emax_v[pl.ds(i * L, L)] = neg
            denom_v[pl.ds(i * L, L)] = zero
            return 0
        lax.fori_loop(0, np_ // L, init_body, 0)
        for buf in (sidxb0, sidxb1, didxb0, didxb1):
            buf[pl.ds(0, L)] = zeroi
            buf[pl.ds(L, L)] = zeroi
        sidxt[...] = zeroi
        didxt[...] = zeroi

        def phase_a_blk(b, _):
            base = s * ea + b * EB
            pltpu.sync_copy(dst_hbm.at[pl.ds(base, EB)], dblk_v)
            pltpu.sync_copy(e_hbm.at[pl.ds(base, EB)], eblk_v)

            def inner(i, _):
                d16 = dblk_v[pl.ds(i * L, L)]
                ev = eblk_v[pl.ds(i * L, L)]
                gmax = _group_max(d16, ev)
                cur = plsc.load_gather(emax_v, [d16])
                plsc.store_scatter(emax_v, [d16], jnp.maximum(cur, gmax))
                return 0
            lax.fori_loop(0, EB // L, inner, 0)
            return 0
        lax.fori_loop(0, ea // EB, phase_a_blk, 0)

        def stripe_reduce(node_v, op):
            pltpu.sync_copy(node_v,
                            red_hbm.at[pl.ds((c * NSUB + s) * np_, np_)])
            plsc.subcore_barrier()
            pltpu.sync_copy(red_hbm.at[pl.ds(c * NSUB * np_ + s * sw, sw)],
                            racc_v)
            for k in range(1, NSUB):
                pltpu.sync_copy(
                    red_hbm.at[pl.ds((c * NSUB + k) * np_ + s * sw, sw)],
                    strip_v)

                def red_body(i, _):
                    sl = pl.ds(i * L, L)
                    racc_v[sl] = op(racc_v[sl], strip_v[sl])
                    return 0
                lax.fori_loop(0, sw // L, red_body, 0)
            pltpu.sync_copy(racc_v, res_hbm.at[pl.ds(c * np_ + s * sw, sw)])
            plsc.subcore_barrier()
            pltpu.sync_copy(res_hbm.at[pl.ds(c * np_, np_)], node_v)

        stripe_reduce(emax_v, jnp.maximum)

        for r in range(B2):
            for f0 in range(d // L):
                rows1a[r, pl.ds(f0 * L, L)] = zero
        for k in range(sw // B2):
            pltpu.sync_copy(rows1a, acc_sh.at[pl.ds(s * sw + k * B2, B2), :])

        def phase_b_blk(b, _):
            base = s * ea + b * EB
            pltpu.sync_copy(dst_hbm.at[pl.ds(base, EB)], dblk_v)
            pltpu.sync_copy(e_hbm.at[pl.ds(base, EB)], eblk_v)

            def inner(i, _):
                d16 = dblk_v[pl.ds(i * L, L)]
                em = plsc.load_gather(emax_v, [d16])
                ex = jnp.exp(eblk_v[pl.ds(i * L, L)] - em)
                eblk_v[pl.ds(i * L, L)] = ex
                gsum, first = _group_sum_first(d16, ex)
                cur = plsc.load_gather(denom_v, [d16])
                plsc.store_scatter(denom_v, [d16], cur + gsum, mask=first)
                return 0
            lax.fori_loop(0, EB // L, inner, 0)
            pltpu.sync_copy(eblk_v, ex_hbm.at[pl.ds(base, EB)])
            return 0
        lax.fori_loop(0, ea // EB, phase_b_blk, 0)

        stripe_reduce(denom_v, jnp.add)

        sb_bufs = (sidxb0, sidxb1)
        db_bufs = (didxb0, didxb1)
        r1_bufs = (rows1a, rows1b)
        r2_bufs = (rows2a, rows2b)
        s1_sems = (sem1a, sem1b)
        s2_sems = (sem2a, sem2b)

        def issue(q, p):
            off = q * B2
            sb, db = sb_bufs[p], db_bufs[p]
            for o in (0, L):
                sb[pl.ds(o, L)] = sblk_v[pl.ds(off + o, L)]
                db[pl.ds(o, L)] = dblk_v[pl.ds(off + o, L)]
            pltpu.async_copy(f1_hbm.at[sb], r1_bufs[p], s1_sems[p])
            pltpu.async_copy(f2_hbm.at[sb], r2_bufs[p], s2_sems[p])

        def combine_half(r1, r2, off, r0):
            d16 = dblk_v[pl.ds(off, L)]
            dn = plsc.load_gather(denom_v, [d16])
            av = eblk_v[pl.ds(off, L)] / dn
            zsplat = jnp.zeros((L,), i32)

            def row_body(j, _):
                ascv = _lane_gather(av, zsplat + j)
                r = r0 + j
                for f0 in range(d // L):
                    sl = pl.ds(f0 * L, L)
                    r1[r, sl] = r1[r, sl] + r2[r, sl] * ascv
                return 0
            lax.fori_loop(0, L, row_body, 0)

        def process(q, p):
            sb, db = sb_bufs[p], db_bufs[p]
            r1, r2 = r1_bufs[p], r2_bufs[p]
            pltpu.make_async_copy(f1_hbm.at[sb], r1, s1_sems[p]).wait()
            pltpu.make_async_copy(f2_hbm.at[sb], r2, s2_sems[p]).wait()
            off = q * B2
            combine_half(r1, r2, off, 0)
            combine_half(r1, r2, off + L, L)
            pltpu.sync_copy(r1, acc_sh.at[db], add=True)

        def phase_c_blk(b, _):
            base = s * ea + c * ec + b * EB
            pltpu.sync_copy(src_hbm.at[pl.ds(base, EB)], sblk_v)
            pltpu.sync_copy(dst_hbm.at[pl.ds(base, EB)], dblk_v)
            pltpu.sync_copy(ex_hbm.at[pl.ds(base, EB)], eblk_v)
            issue(0, 0)
            issue(1, 1)

            def step(m, _):
                process(2 * m, 0)
                issue(2 * m + 2, 0)
                process(2 * m + 1, 1)
                issue(2 * m + 3, 1)
                return 0
            lax.fori_loop(0, NB // 2 - 1, step, 0)
            process(NB - 2, 0)
            process(NB - 1, 1)

            toff = NB * B2
            sidxt[...] = sblk_v[pl.ds(toff, L)]
            didxt[...] = dblk_v[pl.ds(toff, L)]
            r1t = rows1a.at[pl.ds(0, L), :]
            r2t = rows2a.at[pl.ds(0, L), :]
            pltpu.async_copy(f1_hbm.at[sidxt], r1t, sem1a).wait()
            pltpu.async_copy(f2_hbm.at[sidxt], r2t, sem2a).wait()
            combine_half(rows1a, rows2a, toff, 0)
            pltpu.sync_copy(r1t, acc_sh.at[didxt], add=True)
            return 0
        lax.fori_loop(0, ec // EB, phase_c_blk, 0)

        plsc.subcore_barrier()

        @pl.when(c == 0)
        def _():
            pltpu.sync_copy(acc_sh.at[pl.ds(s * sw, sw), :],
                            p0_hbm.at[pl.ds(s * sw, sw), :])

        @pl.when(c == 1)
        def _():
            pltpu.sync_copy(acc_sh.at[pl.ds(s * sw, sw), :],
                            p1_hbm.at[pl.ds(s * sw, sw), :])

    return sc_stage


def kernel(x, edge_index, W_src, W_dst, W_src2, W_dst2, attn_l, attn_r):
    n, d = x.shape
    e_cnt = edge_index.shape[1]
    blk = NSUB * L
    np_ = ((n + blk - 1) // blk) * blk
    sw = np_ // NSUB
    ea = e_cnt // NSUB
    ec = e_cnt // (NSUB * NCORE)

    f1, f2, fd1, el, er = _dense_stage(x, W_src, W_src2, W_dst, W_dst2,
                                       attn_l, attn_r)
    src = edge_index[0]
    dst = edge_index[1]
    e_stage = _make_e_stage(n, e_cnt, np_)
    el_p = jnp.pad(el.reshape(-1), (0, np_ - n))
    er_p = jnp.pad(er.reshape(-1), (0, np_ - n))
    e_vals = e_stage(src, dst, el_p, er_p)
    sc_stage = _make_sc_stage(n, e_cnt, d, np_, sw, ea, ec)
    p0, p1, _ex, _red, _res = sc_stage(src, dst, e_vals, f1, f2)
    return _final_stage(fd1, p0[:n], p1[:n])


# speedup vs baseline: 16.1068x; 1.0876x over previous
import functools

import jax
import jax.numpy as jnp
import numpy as np
from jax import lax
from jax.experimental import pallas as pl
from jax.experimental.pallas import tpu as pltpu
from jax.experimental.pallas import tpu_sc as plsc

L = 16
NSUB = 16
NCORE = 2
EB = 2000
NEG_BIG = -3.0e38


def _mm_body(x_ref, ws_ref, ws2_ref, wd_ref, wd2_ref, al_ref, ar_ref,
             f1_ref, f2_ref, fd1_ref, el_ref, er_ref, elm_ref):
    x = x_ref[...]
    dn = (((1,), (1,)), ((), ()))
    f1 = lax.dot_general(x, ws_ref[...], dn, preferred_element_type=jnp.float32)
    f2 = lax.dot_general(x, ws2_ref[...], dn, preferred_element_type=jnp.float32)
    fd1 = lax.dot_general(x, wd_ref[...], dn, preferred_element_type=jnp.float32)
    fd2 = lax.dot_general(x, wd2_ref[...], dn, preferred_element_type=jnp.float32)
    f1_ref[...] = f1
    f2_ref[...] = f2
    fd1_ref[...] = fd1
    el = lax.dot_general(f2, al_ref[...], dn,
                         preferred_element_type=jnp.float32)
    el_ref[...] = el
    er_ref[...] = lax.dot_general(fd2, ar_ref[...], dn,
                                  preferred_element_type=jnp.float32)

    @pl.when(pl.program_id(0) == 0)
    def _():
        elm_ref[...] = jnp.full((1, 128), NEG_BIG, jnp.float32)
    elm_ref[...] = jnp.maximum(elm_ref[...], jnp.max(el))


def _dense_stage(x, W_src, W_src2, W_dst, W_dst2, attn_l, attn_r):
    n, d = x.shape
    m = 1000
    bs_x = pl.BlockSpec((m, d), lambda i: (i, 0))
    bs_w = pl.BlockSpec((d, d), lambda i: (0, 0))
    bs_a = pl.BlockSpec((1, d), lambda i: (0, 0))
    bs_o = pl.BlockSpec((m, d), lambda i: (i, 0))
    bs_s = pl.BlockSpec((m, 1), lambda i: (i, 0))
    f32 = jnp.float32
    return pl.pallas_call(
        _mm_body,
        grid=(n // m,),
        in_specs=[bs_x, bs_w, bs_w, bs_w, bs_w, bs_a, bs_a],
        out_specs=[bs_o, bs_o, bs_o, bs_s, bs_s,
                   pl.BlockSpec((1, d), lambda i: (0, 0))],
        out_shape=[jax.ShapeDtypeStruct((n, d), f32),
                   jax.ShapeDtypeStruct((n, d), f32),
                   jax.ShapeDtypeStruct((n, d), f32),
                   jax.ShapeDtypeStruct((n, 1), f32),
                   jax.ShapeDtypeStruct((n, 1), f32),
                   jax.ShapeDtypeStruct((1, d), f32)],
    )(x, W_src, W_src2, W_dst, W_dst2, attn_l, attn_r)


def _add3_body(a_ref, b_ref, c_ref, o_ref):
    o_ref[...] = a_ref[...] + b_ref[...] + c_ref[...]


def _final_stage(fd1, p0, p1):
    n, d = fd1.shape
    m = 1000
    bs = pl.BlockSpec((m, d), lambda i: (i, 0))
    return pl.pallas_call(
        _add3_body,
        grid=(n // m,),
        in_specs=[bs, bs, bs],
        out_specs=bs,
        out_shape=jax.ShapeDtypeStruct((n, d), jnp.float32),
    )(fd1, p0, p1)


def _leaky(v):
    return jnp.where(v >= 0.0, v, 0.2 * v)


_GDN = lax.GatherDimensionNumbers(offset_dims=(), collapsed_slice_dims=(0,),
                                  start_index_map=(0,))


def _lane_gather(v, idx):
    return lax.gather(v, idx[:, None], _GDN, (1,),
                      mode=lax.GatherScatterMode.PROMISE_IN_BOUNDS)


def _lane_iota():
    return lax.iota(jnp.int32, L)


def _group_sum_first(d16, ev):
    lane = _lane_iota()
    gsum = ev
    first = lane >= 0
    for k in range(1, L):
        rot = lax.rem(lane + k, L)
        eq = _lane_gather(d16, rot) == d16
        gsum = gsum + jnp.where(eq, _lane_gather(ev, rot), 0.0)
        sh = jnp.maximum(lane - k, 0)
        dup = (_lane_gather(d16, sh) == d16) & (lane >= k)
        first = first & jnp.logical_not(dup)
    return gsum, first


def _make_e_stage(n, e_cnt, np_, d):
    f32 = jnp.float32
    i32 = jnp.int32
    ek = e_cnt // (NSUB * NCORE)
    mesh = plsc.VectorSubcoreMesh(core_axis_name="c", subcore_axis_name="s")

    @functools.partial(
        pl.kernel, mesh=mesh,
        compiler_params=pltpu.CompilerParams(needs_layout_passes=False),
        out_type=jax.ShapeDtypeStruct((e_cnt,), f32),
        scratch_types=[
            pltpu.VMEM((np_,), f32),
            pltpu.VMEM((np_,), f32),
            pltpu.VMEM((d,), f32),
            pltpu.VMEM((EB,), i32),
            pltpu.VMEM((EB,), i32),
            pltpu.VMEM((EB,), f32),
        ])
    def e_stage(src_hbm, dst_hbm, el_hbm, er_hbm, elm_hbm, e_out_hbm,
                el_v, er_v, elm_v, sblk_v, dblk_v, eblk_v):
        c = lax.axis_index("c")
        s = lax.axis_index("s")
        wid = s * NCORE + c
        pltpu.sync_copy(el_hbm, el_v)
        pltpu.sync_copy(er_hbm, er_v)
        pltpu.sync_copy(elm_hbm, elm_v)
        elmax = elm_v[pl.ds(0, L)][0]

        def blk(b, _):
            base = wid * ek + b * EB
            pltpu.sync_copy(src_hbm.at[pl.ds(base, EB)], sblk_v)
            pltpu.sync_copy(dst_hbm.at[pl.ds(base, EB)], dblk_v)

            def inner(i, _):
                s16 = sblk_v[pl.ds(i * L, L)]
                d16 = dblk_v[pl.ds(i * L, L)]
                t16 = plsc.load_gather(er_v, [d16])
                ev = _leaky(plsc.load_gather(el_v, [s16]) + t16)
                bnd = _leaky(elmax + t16)
                eblk_v[pl.ds(i * L, L)] = jnp.exp(ev - bnd)
                return 0
            lax.fori_loop(0, EB // L, inner, 0)
            pltpu.sync_copy(eblk_v, e_out_hbm.at[pl.ds(base, EB)])
            return 0
        lax.fori_loop(0, ek // EB, blk, 0)

    return e_stage


def _make_sc_stage(n, e_cnt, d, np_, sw, ea, ec):
    f32 = jnp.float32
    i32 = jnp.int32
    mesh = plsc.VectorSubcoreMesh(core_axis_name="c", subcore_axis_name="s")

    B2 = 32
    NB = EB // B2
    TL = EB - NB * B2

    @functools.partial(
        pl.kernel, mesh=mesh,
        compiler_params=pltpu.CompilerParams(needs_layout_passes=False),
        out_type=[jax.ShapeDtypeStruct((np_, d), f32),
                  jax.ShapeDtypeStruct((np_, d), f32),
                  jax.ShapeDtypeStruct((NCORE * NSUB * np_,), f32),
                  jax.ShapeDtypeStruct((NCORE * np_,), f32)],
        scratch_types=[
            pltpu.VMEM((np_,), f32),
            pltpu.VMEM((EB,), i32),
            pltpu.VMEM((EB,), i32),
            pltpu.VMEM((EB,), f32),
            pltpu.VMEM((sw,), f32),
            pltpu.VMEM((sw,), f32),
            pltpu.VMEM((B2,), i32),
            pltpu.VMEM((B2,), i32),
            pltpu.VMEM((B2,), i32),
            pltpu.VMEM((B2,), i32),
            pltpu.VMEM((L,), i32),
            pltpu.VMEM((L,), i32),
            pltpu.VMEM((B2, d), f32),
            pltpu.VMEM((B2, d), f32),
            pltpu.VMEM((B2, d), f32),
            pltpu.VMEM((B2, d), f32),
            pltpu.VMEM_SHARED((np_, d), f32),
            pltpu.SemaphoreType.DMA,
            pltpu.SemaphoreType.DMA,
            pltpu.SemaphoreType.DMA,
            pltpu.SemaphoreType.DMA,
        ])
    def sc_stage(src_hbm, dst_hbm, ex_hbm, f1_hbm, f2_hbm,
                 p0_hbm, p1_hbm, red_hbm, res_hbm,
                 denom_v, sblk_v, dblk_v, eblk_v,
                 strip_v, racc_v, sidxb0, sidxb1, didxb0, didxb1,
                 sidxt, didxt, rows1a, rows1b, rows2a, rows2b,
                 acc_sh, sem1a, sem2a, sem1b, sem2b):
        c = lax.axis_index("c")
        s = lax.axis_index("s")

        zero = jnp.zeros((L,), f32)
        zeroi = jnp.zeros((L,), i32)

        def init_body(i, _):
            denom_v[pl.ds(i * L, L)] = zero
            return 0
        lax.fori_loop(0, np_ // L, init_body, 0)
        for buf in (sidxb0, sidxb1, didxb0, didxb1):
            buf[pl.ds(0, L)] = zeroi
            buf[pl.ds(L, L)] = zeroi
        sidxt[...] = zeroi
        didxt[...] = zeroi

        def stripe_reduce(node_v, op):
            pltpu.sync_copy(node_v,
                            red_hbm.at[pl.ds((c * NSUB + s) * np_, np_)])
            plsc.subcore_barrier()
            pltpu.sync_copy(red_hbm.at[pl.ds(c * NSUB * np_ + s * sw, sw)],
                            racc_v)
            for k in range(1, NSUB):
                pltpu.sync_copy(
                    red_hbm.at[pl.ds((c * NSUB + k) * np_ + s * sw, sw)],
                    strip_v)

                def red_body(i, _):
                    sl = pl.ds(i * L, L)
                    racc_v[sl] = op(racc_v[sl], strip_v[sl])
                    return 0
                lax.fori_loop(0, sw // L, red_body, 0)
            pltpu.sync_copy(racc_v, res_hbm.at[pl.ds(c * np_ + s * sw, sw)])
            plsc.subcore_barrier()
            pltpu.sync_copy(res_hbm.at[pl.ds(c * np_, np_)], node_v)

        for r in range(B2):
            for f0 in range(d // L):
                rows1a[r, pl.ds(f0 * L, L)] = zero
        for k in range(sw // B2):
            pltpu.sync_copy(rows1a, acc_sh.at[pl.ds(s * sw + k * B2, B2), :])

        def phase_b_blk(b, _):
            base = s * ea + b * EB
            pltpu.sync_copy(dst_hbm.at[pl.ds(base, EB)], dblk_v)
            pltpu.sync_copy(ex_hbm.at[pl.ds(base, EB)], eblk_v)

            def inner(i, _):
                d16 = dblk_v[pl.ds(i * L, L)]
                ex = eblk_v[pl.ds(i * L, L)]
                gsum, first = _group_sum_first(d16, ex)
                cur = plsc.load_gather(denom_v, [d16])
                plsc.store_scatter(denom_v, [d16], cur + gsum, mask=first)
                return 0
            lax.fori_loop(0, EB // L, inner, 0)
            return 0
        lax.fori_loop(0, ea // EB, phase_b_blk, 0)

        stripe_reduce(denom_v, jnp.add)

        sb_bufs = (sidxb0, sidxb1)
        db_bufs = (didxb0, didxb1)
        r1_bufs = (rows1a, rows1b)
        r2_bufs = (rows2a, rows2b)
        s1_sems = (sem1a, sem1b)
        s2_sems = (sem2a, sem2b)

        def issue(q, p):
            off = q * B2
            sb, db = sb_bufs[p], db_bufs[p]
            for o in (0, L):
                sb[pl.ds(o, L)] = sblk_v[pl.ds(off + o, L)]
                db[pl.ds(o, L)] = dblk_v[pl.ds(off + o, L)]
            pltpu.async_copy(f1_hbm.at[sb], r1_bufs[p], s1_sems[p])
            pltpu.async_copy(f2_hbm.at[sb], r2_bufs[p], s2_sems[p])

        def combine_half(r1, r2, off, r0):
            d16 = dblk_v[pl.ds(off, L)]
            dn = plsc.load_gather(denom_v, [d16])
            av = eblk_v[pl.ds(off, L)] / dn
            zsplat = jnp.zeros((L,), i32)

            def row_body(j, _):
                ascv = _lane_gather(av, zsplat + j)
                r = r0 + j
                for f0 in range(d // L):
                    sl = pl.ds(f0 * L, L)
                    r1[r, sl] = r1[r, sl] + r2[r, sl] * ascv
                return 0
            lax.fori_loop(0, L, row_body, 0)

        def process(q, p):
            sb, db = sb_bufs[p], db_bufs[p]
            r1, r2 = r1_bufs[p], r2_bufs[p]
            pltpu.make_async_copy(f1_hbm.at[sb], r1, s1_sems[p]).wait()
            pltpu.make_async_copy(f2_hbm.at[sb], r2, s2_sems[p]).wait()
            off = q * B2
            combine_half(r1, r2, off, 0)
            combine_half(r1, r2, off + L, L)
            pltpu.sync_copy(r1, acc_sh.at[db], add=True)

        def phase_c_blk(b, _):
            base = s * ea + c * ec + b * EB
            pltpu.sync_copy(src_hbm.at[pl.ds(base, EB)], sblk_v)
            pltpu.sync_copy(dst_hbm.at[pl.ds(base, EB)], dblk_v)
            pltpu.sync_copy(ex_hbm.at[pl.ds(base, EB)], eblk_v)
            issue(0, 0)
            issue(1, 1)

            def step(m, _):
                process(2 * m, 0)
                issue(2 * m + 2, 0)
                process(2 * m + 1, 1)
                issue(2 * m + 3, 1)
                return 0
            lax.fori_loop(0, NB // 2 - 1, step, 0)
            process(NB - 2, 0)
            process(NB - 1, 1)

            toff = NB * B2
            sidxt[...] = sblk_v[pl.ds(toff, L)]
            didxt[...] = dblk_v[pl.ds(toff, L)]
            r1t = rows1a.at[pl.ds(0, L), :]
            r2t = rows2a.at[pl.ds(0, L), :]
            pltpu.async_copy(f1_hbm.at[sidxt], r1t, sem1a).wait()
            pltpu.async_copy(f2_hbm.at[sidxt], r2t, sem2a).wait()
            combine_half(rows1a, rows2a, toff, 0)
            pltpu.sync_copy(r1t, acc_sh.at[didxt], add=True)
            return 0
        lax.fori_loop(0, ec // EB, phase_c_blk, 0)

        plsc.subcore_barrier()

        @pl.when(c == 0)
        def _():
            pltpu.sync_copy(acc_sh.at[pl.ds(s * sw, sw), :],
                            p0_hbm.at[pl.ds(s * sw, sw), :])

        @pl.when(c == 1)
        def _():
            pltpu.sync_copy(acc_sh.at[pl.ds(s * sw, sw), :],
                            p1_hbm.at[pl.ds(s * sw, sw), :])

    return sc_stage


def kernel(x, edge_index, W_src, W_dst, W_src2, W_dst2, attn_l, attn_r):
    n, d = x.shape
    e_cnt = edge_index.shape[1]
    blk = NSUB * L
    np_ = ((n + blk - 1) // blk) * blk
    sw = np_ // NSUB
    ea = e_cnt // NSUB
    ec = e_cnt // (NSUB * NCORE)

    f1, f2, fd1, el, er, elm = _dense_stage(x, W_src, W_src2, W_dst, W_dst2,
                                            attn_l, attn_r)
    src = edge_index[0]
    dst = edge_index[1]
    e_stage = _make_e_stage(n, e_cnt, np_, d)
    el_p = jnp.pad(el.reshape(-1), (0, np_ - n))
    er_p = jnp.pad(er.reshape(-1), (0, np_ - n))
    ex_vals = e_stage(src, dst, el_p, er_p, elm.reshape(-1))
    sc_stage = _make_sc_stage(n, e_cnt, d, np_, sw, ea, ec)
    p0, p1, _red, _res = sc_stage(src, dst, ex_vals, f1, f2)
    return _final_stage(fd1, p0[:n], p1[:n])


# speedup vs baseline: 17.0261x; 1.0571x over previous
import functools

import jax
import jax.numpy as jnp
import numpy as np
from jax import lax
from jax.experimental import pallas as pl
from jax.experimental.pallas import tpu as pltpu
from jax.experimental.pallas import tpu_sc as plsc

L = 16
NSUB = 16
NCORE = 2
EB = 2000
NEG_BIG = -3.0e38


def _mm_body(x_ref, ws_ref, ws2_ref, wd_ref, wd2_ref, al_ref, ar_ref,
             f1_ref, f2_ref, fd1_ref, el_ref, er_ref, elm_ref):
    x = x_ref[...]
    dn = (((1,), (1,)), ((), ()))
    f1 = lax.dot_general(x, ws_ref[...], dn, preferred_element_type=jnp.float32)
    f2 = lax.dot_general(x, ws2_ref[...], dn, preferred_element_type=jnp.float32)
    fd1 = lax.dot_general(x, wd_ref[...], dn, preferred_element_type=jnp.float32)
    fd2 = lax.dot_general(x, wd2_ref[...], dn, preferred_element_type=jnp.float32)
    f1_ref[...] = f1
    f2_ref[...] = f2
    fd1_ref[...] = fd1
    el = lax.dot_general(f2, al_ref[...], dn,
                         preferred_element_type=jnp.float32)
    el_ref[...] = el
    er_ref[...] = lax.dot_general(fd2, ar_ref[...], dn,
                                  preferred_element_type=jnp.float32)

    @pl.when(pl.program_id(0) == 0)
    def _():
        elm_ref[...] = jnp.full((1, 128), NEG_BIG, jnp.float32)
    elm_ref[...] = jnp.maximum(elm_ref[...], jnp.max(el))


def _dense_stage(x, W_src, W_src2, W_dst, W_dst2, attn_l, attn_r):
    n, d = x.shape
    m = 1000
    bs_x = pl.BlockSpec((m, d), lambda i: (i, 0))
    bs_w = pl.BlockSpec((d, d), lambda i: (0, 0))
    bs_a = pl.BlockSpec((1, d), lambda i: (0, 0))
    bs_o = pl.BlockSpec((m, d), lambda i: (i, 0))
    bs_s = pl.BlockSpec((m, 1), lambda i: (i, 0))
    f32 = jnp.float32
    return pl.pallas_call(
        _mm_body,
        grid=(n // m,),
        in_specs=[bs_x, bs_w, bs_w, bs_w, bs_w, bs_a, bs_a],
        out_specs=[bs_o, bs_o, bs_o, bs_s, bs_s,
                   pl.BlockSpec((1, d), lambda i: (0, 0))],
        out_shape=[jax.ShapeDtypeStruct((n, d), f32),
                   jax.ShapeDtypeStruct((n, d), f32),
                   jax.ShapeDtypeStruct((n, d), f32),
                   jax.ShapeDtypeStruct((n, 1), f32),
                   jax.ShapeDtypeStruct((n, 1), f32),
                   jax.ShapeDtypeStruct((1, d), f32)],
    )(x, W_src, W_src2, W_dst, W_dst2, attn_l, attn_r)


def _add3_body(a_ref, b_ref, c_ref, o_ref):
    o_ref[...] = a_ref[...] + b_ref[...] + c_ref[...]


def _final_stage(fd1, p0, p1):
    n, d = fd1.shape
    m = 1000
    bs = pl.BlockSpec((m, d), lambda i: (i, 0))
    return pl.pallas_call(
        _add3_body,
        grid=(n // m,),
        in_specs=[bs, bs, bs],
        out_specs=bs,
        out_shape=jax.ShapeDtypeStruct((n, d), jnp.float32),
    )(fd1, p0, p1)


def _leaky(v):
    return jnp.where(v >= 0.0, v, 0.2 * v)


_GDN = lax.GatherDimensionNumbers(offset_dims=(), collapsed_slice_dims=(0,),
                                  start_index_map=(0,))


def _lane_gather(v, idx):
    return lax.gather(v, idx[:, None], _GDN, (1,),
                      mode=lax.GatherScatterMode.PROMISE_IN_BOUNDS)


def _lane_iota():
    return lax.iota(jnp.int32, L)


def _group_sum_first(d16, ev):
    lane = _lane_iota()
    gsum = ev
    first = lane >= 0
    for k in range(1, L):
        rot = lax.rem(lane + k, L)
        eq = _lane_gather(d16, rot) == d16
        gsum = gsum + jnp.where(eq, _lane_gather(ev, rot), 0.0)
        sh = jnp.maximum(lane - k, 0)
        dup = (_lane_gather(d16, sh) == d16) & (lane >= k)
        first = first & jnp.logical_not(dup)
    return gsum, first


def _make_e_stage(n, e_cnt, np_, d):
    f32 = jnp.float32
    i32 = jnp.int32
    ek = e_cnt // (NSUB * NCORE)
    mesh = plsc.VectorSubcoreMesh(core_axis_name="c", subcore_axis_name="s")

    @functools.partial(
        pl.kernel, mesh=mesh,
        compiler_params=pltpu.CompilerParams(needs_layout_passes=False),
        out_type=jax.ShapeDtypeStruct((e_cnt,), f32),
        scratch_types=[
            pltpu.VMEM((np_,), f32),
            pltpu.VMEM((np_,), f32),
            pltpu.VMEM((d,), f32),
            pltpu.VMEM((EB,), i32),
            pltpu.VMEM((EB,), i32),
            pltpu.VMEM((EB,), f32),
        ])
    def e_stage(src_hbm, dst_hbm, el_hbm, er_hbm, elm_hbm, e_out_hbm,
                el_v, er_v, elm_v, sblk_v, dblk_v, eblk_v):
        c = lax.axis_index("c")
        s = lax.axis_index("s")
        wid = s * NCORE + c
        pltpu.sync_copy(el_hbm, el_v)
        pltpu.sync_copy(er_hbm, er_v)
        pltpu.sync_copy(elm_hbm, elm_v)
        elmax = elm_v[pl.ds(0, L)][0]

        def blk(b, _):
            base = wid * ek + b * EB
            pltpu.sync_copy(src_hbm.at[pl.ds(base, EB)], sblk_v)
            pltpu.sync_copy(dst_hbm.at[pl.ds(base, EB)], dblk_v)

            def inner(i, _):
                s16 = sblk_v[pl.ds(i * L, L)]
                d16 = dblk_v[pl.ds(i * L, L)]
                t16 = plsc.load_gather(er_v, [d16])
                ev = _leaky(plsc.load_gather(el_v, [s16]) + t16)
                bnd = _leaky(elmax + t16)
                eblk_v[pl.ds(i * L, L)] = jnp.exp(ev - bnd)
                return 0
            lax.fori_loop(0, EB // L, inner, 0)
            pltpu.sync_copy(eblk_v, e_out_hbm.at[pl.ds(base, EB)])
            return 0
        lax.fori_loop(0, ek // EB, blk, 0)

    return e_stage


def _make_sc_stage(n, e_cnt, d, np_, sw, ea, ec):
    f32 = jnp.float32
    i32 = jnp.int32
    mesh = plsc.VectorSubcoreMesh(core_axis_name="c", subcore_axis_name="s")

    B2 = 32
    NB = EB // B2
    TL = EB - NB * B2

    @functools.partial(
        pl.kernel, mesh=mesh,
        compiler_params=pltpu.CompilerParams(needs_layout_passes=False),
        out_type=[jax.ShapeDtypeStruct((np_, d), f32),
                  jax.ShapeDtypeStruct((np_, d), f32),
                  jax.ShapeDtypeStruct((NCORE * NSUB * np_,), f32),
                  jax.ShapeDtypeStruct((NCORE * np_,), f32)],
        scratch_types=[
            pltpu.VMEM((np_,), f32),
            pltpu.VMEM((EB,), i32),
            pltpu.VMEM((EB,), i32),
            pltpu.VMEM((EB,), f32),
            pltpu.VMEM((sw,), f32),
            pltpu.VMEM((sw,), f32),
            pltpu.VMEM((B2,), i32),
            pltpu.VMEM((B2,), i32),
            pltpu.VMEM((B2,), i32),
            pltpu.VMEM((B2,), i32),
            pltpu.VMEM((L,), i32),
            pltpu.VMEM((L,), i32),
            pltpu.VMEM((B2, d), f32),
            pltpu.VMEM((B2, d), f32),
            pltpu.VMEM((B2, d), f32),
            pltpu.VMEM((B2, d), f32),
            pltpu.VMEM_SHARED((np_, d), f32),
            pltpu.SemaphoreType.DMA,
            pltpu.SemaphoreType.DMA,
            pltpu.SemaphoreType.DMA,
            pltpu.SemaphoreType.DMA,
        ])
    def sc_stage(src_hbm, dst_hbm, ex_hbm, f1_hbm, f2_hbm,
                 p0_hbm, p1_hbm, red_hbm, res_hbm,
                 denom_v, sblk_v, dblk_v, eblk_v,
                 strip_v, racc_v, sidxb0, sidxb1, didxb0, didxb1,
                 sidxt, didxt, rows1a, rows1b, rows2a, rows2b,
                 acc_sh, sem1a, sem2a, sem1b, sem2b):
        c = lax.axis_index("c")
        s = lax.axis_index("s")

        zero = jnp.zeros((L,), f32)
        zeroi = jnp.zeros((L,), i32)

        def init_body(i, _):
            denom_v[pl.ds(i * L, L)] = zero
            return 0
        lax.fori_loop(0, np_ // L, init_body, 0)
        for buf in (sidxb0, sidxb1, didxb0, didxb1):
            buf[pl.ds(0, L)] = zeroi
            buf[pl.ds(L, L)] = zeroi
        sidxt[...] = zeroi
        didxt[...] = zeroi

        def stripe_reduce(node_v, op):
            pltpu.sync_copy(node_v,
                            red_hbm.at[pl.ds((c * NSUB + s) * np_, np_)])
            plsc.subcore_barrier()
            pltpu.sync_copy(red_hbm.at[pl.ds(c * NSUB * np_ + s * sw, sw)],
                            racc_v)
            for k in range(1, NSUB):
                pltpu.sync_copy(
                    red_hbm.at[pl.ds((c * NSUB + k) * np_ + s * sw, sw)],
                    strip_v)

                def red_body(i, _):
                    sl = pl.ds(i * L, L)
                    racc_v[sl] = op(racc_v[sl], strip_v[sl])
                    return 0
                lax.fori_loop(0, sw // L, red_body, 0)
            pltpu.sync_copy(racc_v, res_hbm.at[pl.ds(c * np_ + s * sw, sw)])
            plsc.subcore_barrier()
            pltpu.sync_copy(res_hbm.at[pl.ds(c * np_, np_)], node_v)

        for r in range(B2):
            for f0 in range(d // L):
                rows1a[r, pl.ds(f0 * L, L)] = zero
        for k in range(sw // B2):
            pltpu.sync_copy(rows1a, acc_sh.at[pl.ds(s * sw + k * B2, B2), :])

        def phase_b_blk(b, _):
            base = s * ea + b * EB
            pltpu.sync_copy(dst_hbm.at[pl.ds(base, EB)], dblk_v)
            pltpu.sync_copy(ex_hbm.at[pl.ds(base, EB)], eblk_v)

            def inner(i, _):
                d16 = dblk_v[pl.ds(i * L, L)]
                ex = eblk_v[pl.ds(i * L, L)]
                plsc.addupdate_scatter(denom_v, [d16], ex)
                return 0
            lax.fori_loop(0, EB // L, inner, 0)
            return 0
        lax.fori_loop(0, ea // EB, phase_b_blk, 0)

        stripe_reduce(denom_v, jnp.add)

        sb_bufs = (sidxb0, sidxb1)
        db_bufs = (didxb0, didxb1)
        r1_bufs = (rows1a, rows1b)
        r2_bufs = (rows2a, rows2b)
        s1_sems = (sem1a, sem1b)
        s2_sems = (sem2a, sem2b)

        def issue(q, p):
            off = q * B2
            sb, db = sb_bufs[p], db_bufs[p]
            for o in (0, L):
                sb[pl.ds(o, L)] = sblk_v[pl.ds(off + o, L)]
                db[pl.ds(o, L)] = dblk_v[pl.ds(off + o, L)]
            pltpu.async_copy(f1_hbm.at[sb], r1_bufs[p], s1_sems[p])
            pltpu.async_copy(f2_hbm.at[sb], r2_bufs[p], s2_sems[p])

        def combine_half(r1, r2, off, r0):
            d16 = dblk_v[pl.ds(off, L)]
            dn = plsc.load_gather(denom_v, [d16])
            av = eblk_v[pl.ds(off, L)] / dn
            zsplat = jnp.zeros((L,), i32)

            def row_body(j, _):
                ascv = _lane_gather(av, zsplat + j)
                r = r0 + j
                for f0 in range(d // L):
                    sl = pl.ds(f0 * L, L)
                    r1[r, sl] = r1[r, sl] + r2[r, sl] * ascv
                return 0
            lax.fori_loop(0, L, row_body, 0)

        def process(q, p):
            sb, db = sb_bufs[p], db_bufs[p]
            r1, r2 = r1_bufs[p], r2_bufs[p]
            pltpu.make_async_copy(f1_hbm.at[sb], r1, s1_sems[p]).wait()
            pltpu.make_async_copy(f2_hbm.at[sb], r2, s2_sems[p]).wait()
            off = q * B2
            combine_half(r1, r2, off, 0)
            combine_half(r1, r2, off + L, L)
            pltpu.sync_copy(r1, acc_sh.at[db], add=True)

        def phase_c_blk(b, _):
            base = s * ea + c * ec + b * EB
            pltpu.sync_copy(src_hbm.at[pl.ds(base, EB)], sblk_v)
            pltpu.sync_copy(dst_hbm.at[pl.ds(base, EB)], dblk_v)
            pltpu.sync_copy(ex_hbm.at[pl.ds(base, EB)], eblk_v)
            issue(0, 0)
            issue(1, 1)

            def step(m, _):
                process(2 * m, 0)
                issue(2 * m + 2, 0)
                process(2 * m + 1, 1)
                issue(2 * m + 3, 1)
                return 0
            lax.fori_loop(0, NB // 2 - 1, step, 0)
            process(NB - 2, 0)
            process(NB - 1, 1)

            toff = NB * B2
            sidxt[...] = sblk_v[pl.ds(toff, L)]
            didxt[...] = dblk_v[pl.ds(toff, L)]
            r1t = rows1a.at[pl.ds(0, L), :]
            r2t = rows2a.at[pl.ds(0, L), :]
            pltpu.async_copy(f1_hbm.at[sidxt], r1t, sem1a).wait()
            pltpu.async_copy(f2_hbm.at[sidxt], r2t, sem2a).wait()
            combine_half(rows1a, rows2a, toff, 0)
            pltpu.sync_copy(r1t, acc_sh.at[didxt], add=True)
            return 0
        lax.fori_loop(0, ec // EB, phase_c_blk, 0)

        plsc.subcore_barrier()

        @pl.when(c == 0)
        def _():
            pltpu.sync_copy(acc_sh.at[pl.ds(s * sw, sw), :],
                            p0_hbm.at[pl.ds(s * sw, sw), :])

        @pl.when(c == 1)
        def _():
            pltpu.sync_copy(acc_sh.at[pl.ds(s * sw, sw), :],
                            p1_hbm.at[pl.ds(s * sw, sw), :])

    return sc_stage


def kernel(x, edge_index, W_src, W_dst, W_src2, W_dst2, attn_l, attn_r):
    n, d = x.shape
    e_cnt = edge_index.shape[1]
    blk = NSUB * L
    np_ = ((n + blk - 1) // blk) * blk
    sw = np_ // NSUB
    ea = e_cnt // NSUB
    ec = e_cnt // (NSUB * NCORE)

    f1, f2, fd1, el, er, elm = _dense_stage(x, W_src, W_src2, W_dst, W_dst2,
                                            attn_l, attn_r)
    src = edge_index[0]
    dst = edge_index[1]
    e_stage = _make_e_stage(n, e_cnt, np_, d)
    el_p = jnp.pad(el.reshape(-1), (0, np_ - n))
    er_p = jnp.pad(er.reshape(-1), (0, np_ - n))
    ex_vals = e_stage(src, dst, el_p, er_p, elm.reshape(-1))
    sc_stage = _make_sc_stage(n, e_cnt, d, np_, sw, ea, ec)
    p0, p1, _red, _res = sc_stage(src, dst, ex_vals, f1, f2)
    return _final_stage(fd1, p0[:n], p1[:n])
